# baseline (device time: 13931 ns/iter reference)
import jax
import jax.numpy as jnp
from jax import lax
from jax.experimental import pallas as pl
from jax.experimental.pallas import tpu as pltpu

N_Y = 2
EPS = 1e-5
C = 4


def build(use_comm=True):
    def kernel(x, gamma, beta):
        m, n_loc = x.shape
        n_glob = N_Y * n_loc
        r = m // C
        h = m // 2
        g2 = gamma.reshape(1, n_loc)
        b2 = beta.reshape(1, n_loc)

        def body(
            x_vmem, g_ref, b_ref, out_ref,
            send_buf, recv_buf,
            send_sems, recv_sems,
        ):
            my_x = lax.axis_index("x")
            my_y = lax.axis_index("y")
            nbr = (my_x, 1 - my_y)

            if use_comm:
                barrier_sem = pltpu.get_barrier_semaphore()
                pl.semaphore_signal(
                    barrier_sem, inc=1, device_id=nbr,
                    device_id_type=pl.DeviceIdType.MESH,
                )

            def chunk_stats(c):
                xc = x_vmem[pl.ds(c * r, r), :]
                s = jnp.sum(xc, axis=1)
                ss = jnp.sum(xc * xc, axis=1)
                send_buf[0:1, pl.ds(c * r, r)] = s.reshape(1, r)
                send_buf[1:2, pl.ds(c * r, r)] = ss.reshape(1, r)

            def make_rdma(half):
                return pltpu.make_async_remote_copy(
                    src_ref=send_buf.at[:, pl.ds(half * h, h)],
                    dst_ref=recv_buf.at[:, pl.ds(half * h, h)],
                    send_sem=send_sems.at[half],
                    recv_sem=recv_sems.at[half],
                    device_id=nbr,
                    device_id_type=pl.DeviceIdType.MESH,
                )

            chunk_stats(0)
            chunk_stats(1)
            if use_comm:
                pl.semaphore_wait(barrier_sem, 1)
                rdma_a = make_rdma(0)
                rdma_a.start()
            chunk_stats(2)
            chunk_stats(3)
            if use_comm:
                rdma_b = make_rdma(1)
                rdma_b.start()

            gbf = g_ref[:, :].astype(jnp.bfloat16)
            bbf = b_ref[:, :].astype(jnp.bfloat16)

            def normalize_half(half):
                sl = pl.ds(half * h, h)
                if use_comm:
                    tot_s = send_buf[0:1, sl] + recv_buf[0:1, sl]
                    tot_ss = send_buf[1:2, sl] + recv_buf[1:2, sl]
                else:
                    tot_s = send_buf[0:1, sl] * 2.0
                    tot_ss = send_buf[1:2, sl] * 2.0
                mean = tot_s / n_glob
                var = tot_ss / n_glob - mean * mean
                rstd = lax.rsqrt(var + EPS)
                mr = jnp.concatenate([mean, rstd], axis=1).reshape(2 * h, 1)
                mean_col = mr[0:h, :]
                rstd_col = mr[h : 2 * h, :]
                mean_bf = mean_col.astype(jnp.bfloat16)
                rstd_bf = rstd_col.astype(jnp.bfloat16)
                for k in range(h // r):
                    c = half * (h // r) + k
                    row0 = c * r - half * h
                    xc = x_vmem[pl.ds(c * r, r), :].astype(jnp.bfloat16)
                    t = (
                        xc - mean_bf[row0 : row0 + r, :]
                    ) * rstd_bf[row0 : row0 + r, :]
                    out_ref[pl.ds(c * r, r), :] = t * gbf + bbf

            if use_comm:
                rdma_a.wait()
            normalize_half(0)
            if use_comm:
                rdma_b.wait()
            normalize_half(1)

        return pl.pallas_call(
            body,
            out_shape=jax.ShapeDtypeStruct((m, n_loc), jnp.bfloat16),
            in_specs=[
                pl.BlockSpec(memory_space=pltpu.VMEM),
                pl.BlockSpec(memory_space=pltpu.VMEM),
                pl.BlockSpec(memory_space=pltpu.VMEM),
            ],
            out_specs=pl.BlockSpec(memory_space=pltpu.VMEM),
            scratch_shapes=[
                pltpu.VMEM((2, m), jnp.float32),
                pltpu.VMEM((2, m), jnp.float32),
                pltpu.SemaphoreType.DMA((2,)),
                pltpu.SemaphoreType.DMA((2,)),
            ],
            compiler_params=pltpu.CompilerParams(
                collective_id=0 if use_comm else None
            ),
        )(x, g2, b2)

    return kernel


kernel = build(True)


# device time: 11276 ns/iter; 1.2355x vs baseline; 1.2355x over previous
import jax
import jax.numpy as jnp
from jax import lax
from jax.experimental import pallas as pl
from jax.experimental.pallas import tpu as pltpu

N_Y = 2
EPS = 1e-5
C = 4


def build(use_comm=True):
    def kernel(x, gamma, beta):
        m, n_loc = x.shape
        n_glob = N_Y * n_loc
        r = m // C
        g2 = gamma.reshape(1, n_loc)
        b2 = beta.reshape(1, n_loc)
        xb = x.astype(jnp.bfloat16)

        def body(
            x_vmem, g_ref, b_ref, out_ref,
            send_buf, recv_buf,
            send_sems, recv_sems,
        ):
            my_x = lax.axis_index("x")
            my_y = lax.axis_index("y")
            nbr = (my_x, 1 - my_y)

            if use_comm:
                barrier_sem = pltpu.get_barrier_semaphore()
                pl.semaphore_signal(
                    barrier_sem, inc=1, device_id=nbr,
                    device_id_type=pl.DeviceIdType.MESH,
                )

            def chunk_stats(c):
                xc = x_vmem[pl.ds(c * r, r), :]
                s = jnp.sum(xc, axis=1, dtype=jnp.float32)
                ss = jnp.sum(xc * xc, axis=1, dtype=jnp.float32)
                send_buf[:, pl.ds(c * r, r)] = jnp.stack([s, ss])

            def make_rdma(c):
                return pltpu.make_async_remote_copy(
                    src_ref=send_buf.at[:, pl.ds(c * r, r)],
                    dst_ref=recv_buf.at[:, pl.ds(c * r, r)],
                    send_sem=send_sems.at[c],
                    recv_sem=recv_sems.at[c],
                    device_id=nbr,
                    device_id_type=pl.DeviceIdType.MESH,
                )

            rdmas = []
            for c in range(C):
                chunk_stats(c)
                if use_comm:
                    if c == 0:
                        pl.semaphore_wait(barrier_sem, 1)
                    rd = make_rdma(c)
                    rd.start()
                    rdmas.append(rd)

            gbf = g_ref[:, :].astype(jnp.bfloat16)
            bbf = b_ref[:, :].astype(jnp.bfloat16)

            def normalize_chunk(c):
                sl = pl.ds(c * r, r)
                if use_comm:
                    rdmas[c].wait()
                    tot_s = send_buf[0:1, sl] + recv_buf[0:1, sl]
                    tot_ss = send_buf[1:2, sl] + recv_buf[1:2, sl]
                else:
                    tot_s = send_buf[0:1, sl] * 2.0
                    tot_ss = send_buf[1:2, sl] * 2.0
                mean = tot_s / n_glob
                var = tot_ss / n_glob - mean * mean
                rstd = lax.rsqrt(var + EPS)
                mr = jnp.concatenate([mean, rstd], axis=1).reshape(2 * r, 1)
                mean_bf = mr[0:r, :].astype(jnp.bfloat16)
                rstd_bf = mr[r : 2 * r, :].astype(jnp.bfloat16)
                xc = x_vmem[sl, :]
                t = (xc - mean_bf) * rstd_bf
                out_ref[sl, :] = t * gbf + bbf

            for c in range(C):
                normalize_chunk(c)

        return pl.pallas_call(
            body,
            out_shape=jax.ShapeDtypeStruct((m, n_loc), jnp.bfloat16),
            in_specs=[
                pl.BlockSpec(memory_space=pltpu.VMEM),
                pl.BlockSpec(memory_space=pltpu.VMEM),
                pl.BlockSpec(memory_space=pltpu.VMEM),
            ],
            out_specs=pl.BlockSpec(memory_space=pltpu.VMEM),
            scratch_shapes=[
                pltpu.VMEM((2, m), jnp.float32),
                pltpu.VMEM((2, m), jnp.float32),
                pltpu.SemaphoreType.DMA((C,)),
                pltpu.SemaphoreType.DMA((C,)),
            ],
            compiler_params=pltpu.CompilerParams(
                collective_id=0 if use_comm else None
            ),
        )(xb, g2, b2)

    return kernel


kernel = build(True)
